# baseline (speedup 1.0000x reference)
import jax
import jax.numpy as jnp
from jax.experimental import pallas as pl

B, T, N, H = 8, 512, 400, 128
EPS = 1e-8
_BF = jnp.bfloat16
_F32 = jnp.float32
_CONTRACT0 = (((0,), (0,)), ((), ()))


def _gcn_block_kernel(fmri_ref, w1_ref, b1_ref, w2_ref, b2_ref, out_ref):
    for blk in range(8):
        _one_block(fmri_ref[blk], w1_ref, b1_ref, w2_ref, b2_ref,
                   out_ref, blk)


def _one_block(xb, w1_ref, b1_ref, w2_ref, b2_ref, out_ref, blk):
    gram = jax.lax.dot_general(
        xb, xb, _CONTRACT0, preferred_element_type=_F32)
    x = xb.astype(_F32)
    colsum = jnp.sum(x, axis=0)
    sumsq = jnp.sum(x * x, axis=0)
    m = colsum * (1.0 / T)
    var = sumsq - T * m * m
    s = jnp.sqrt(var)
    inv_s = jnp.where(s > 0.0, 1.0 / s, 0.0)
    u = m * inv_s
    corr = (gram * inv_s[:, None] * inv_s[None, :]
            - T * u[:, None] * u[None, :])
    adj = jnp.abs(jnp.clip(corr, -1.0, 1.0))
    adjb = adj.astype(_BF)
    rs1 = jnp.sum(adj, axis=1, keepdims=True)
    rs2 = jnp.sum(adj * adj, axis=1, keepdims=True)
    inv_rn = 1.0 / (jnp.sqrt(rs2) + EPS)
    deg = rs1 * inv_rn + 1.0
    dinv = jax.lax.rsqrt(deg)

    def layer(sup):
        supd = sup * dinv
        agg = inv_rn * jnp.dot(adjb, supd.astype(_BF),
                               preferred_element_type=_F32)
        return jnp.maximum((agg + supd) * dinv, 0.0)

    w1b = w1_ref[...].astype(_BF)
    s1 = inv_rn * jnp.dot(adjb, w1b, preferred_element_type=_F32)
    x1 = layer(s1 + b1_ref[...])
    s2 = jnp.dot(x1.astype(_BF), w2_ref[...].astype(_BF),
                 preferred_element_type=_F32)
    out_ref[blk] = layer(s2 + b2_ref[...])


@jax.jit
def kernel(fMRI, W1, b1, W2, b2):
    fMRIb = fMRI.astype(_BF)
    b1r = b1.reshape(1, H)
    b2r = b2.reshape(1, H)
    out = pl.pallas_call(
        _gcn_block_kernel,
        grid=(B // 8,),
        in_specs=[
            pl.BlockSpec((8, T, N), lambda b: (b, 0, 0)),
            pl.BlockSpec((N, H), lambda b: (0, 0)),
            pl.BlockSpec((1, H), lambda b: (0, 0)),
            pl.BlockSpec((H, H), lambda b: (0, 0)),
            pl.BlockSpec((1, H), lambda b: (0, 0)),
        ],
        out_specs=pl.BlockSpec((8, N, H), lambda b: (b, 0, 0)),
        out_shape=jax.ShapeDtypeStruct((B, N, H), jnp.float32),
    )(fMRIb, W1, b1r, W2, b2r)
    return out


# speedup vs baseline: 1.0165x; 1.0165x over previous
import jax
import jax.numpy as jnp
from jax.experimental import pallas as pl

B, T, N, H = 8, 512, 400, 128
EPS = 1e-8
_BF = jnp.bfloat16
_F32 = jnp.float32
_CONTRACT0 = (((0,), (0,)), ((), ()))


def _gcn_block_kernel(fmri_ref, w1_ref, b1_ref, w2_ref, b2_ref, out_ref):
    for blk in range(4):
        _one_block(fmri_ref[blk], w1_ref, b1_ref, w2_ref, b2_ref,
                   out_ref, blk)


def _one_block(xb, w1_ref, b1_ref, w2_ref, b2_ref, out_ref, blk):
    gram = jax.lax.dot_general(
        xb, xb, _CONTRACT0, preferred_element_type=_F32)
    x = xb.astype(_F32)
    colsum = jnp.sum(x, axis=0)
    sumsq = jnp.sum(x * x, axis=0)
    m = colsum * (1.0 / T)
    var = sumsq - T * m * m
    s = jnp.sqrt(var)
    inv_s = jnp.where(s > 0.0, 1.0 / s, 0.0)
    u = m * inv_s
    corr = (gram * inv_s[:, None] * inv_s[None, :]
            - T * u[:, None] * u[None, :])
    adj = jnp.abs(jnp.clip(corr, -1.0, 1.0))
    adjb = adj.astype(_BF)
    rs1 = jnp.sum(adj, axis=1, keepdims=True)
    rs2 = jnp.sum(adj * adj, axis=1, keepdims=True)
    inv_rn = 1.0 / (jnp.sqrt(rs2) + EPS)
    deg = rs1 * inv_rn + 1.0
    dinv = jax.lax.rsqrt(deg)

    def layer(sup):
        supd = sup * dinv
        agg = inv_rn * jnp.dot(adjb, supd.astype(_BF),
                               preferred_element_type=_F32)
        return jnp.maximum((agg + supd) * dinv, 0.0)

    w1b = w1_ref[...].astype(_BF)
    s1 = inv_rn * jnp.dot(adjb, w1b, preferred_element_type=_F32)
    x1 = layer(s1 + b1_ref[...])
    s2 = jnp.dot(x1.astype(_BF), w2_ref[...].astype(_BF),
                 preferred_element_type=_F32)
    out_ref[blk] = layer(s2 + b2_ref[...])


@jax.jit
def kernel(fMRI, W1, b1, W2, b2):
    fMRIb = fMRI.astype(_BF)
    b1r = b1.reshape(1, H)
    b2r = b2.reshape(1, H)
    out = pl.pallas_call(
        _gcn_block_kernel,
        grid=(B // 4,),
        in_specs=[
            pl.BlockSpec((4, T, N), lambda b: (b, 0, 0)),
            pl.BlockSpec((N, H), lambda b: (0, 0)),
            pl.BlockSpec((1, H), lambda b: (0, 0)),
            pl.BlockSpec((H, H), lambda b: (0, 0)),
            pl.BlockSpec((1, H), lambda b: (0, 0)),
        ],
        out_specs=pl.BlockSpec((4, N, H), lambda b: (b, 0, 0)),
        out_shape=jax.ShapeDtypeStruct((B, N, H), jnp.float32),
    )(fMRIb, W1, b1r, W2, b2r)
    return out
